# baseline (speedup 1.0000x reference)
import functools

import jax
import jax.numpy as jnp
import numpy as np
from jax.experimental import pallas as pl
from jax.experimental.pallas import tpu as pltpu

_NUM_TABLE = 32
_CODE_LENGTH = 16
_HIDDEN = 2048
_TOTAL = _NUM_TABLE * _CODE_LENGTH
_BM = 1024
_NC = 4
_CW = _TOTAL // _NC


def _fused_kernel(x_ref, w_ref, b_ref, selc_ref, sele_ref, code_ref, score_ref):
    x = x_ref[...]
    codef = jnp.zeros((x.shape[0], _NUM_TABLE), jnp.float32)
    score = jnp.zeros((x.shape[0], _NUM_TABLE), jnp.float32)
    for c in range(_NC):
        lo = c * _CW
        z = jnp.dot(x, w_ref[:, lo:lo + _CW],
                    preferred_element_type=jnp.float32)
        z = z + b_ref[:, lo:lo + _CW]
        bits = (z > 0).astype(jnp.float32)
        codef = codef + jax.lax.dot(bits, selc_ref[lo:lo + _CW, :],
                                    preferred_element_type=jnp.float32)
        za = jnp.abs(z)
        p = za * pltpu.roll(za, _CW - 1, 1)
        p = p * pltpu.roll(p, _CW - 2, 1)
        p = p * pltpu.roll(p, _CW - 4, 1)
        p = p * pltpu.roll(p, _CW - 8, 1)
        score = score + jax.lax.dot(p, sele_ref[lo:lo + _CW, :],
                                    preferred_element_type=jnp.float32)
    code_ref[...] = codef.astype(jnp.int32)
    score_ref[...] = score


@functools.partial(jax.jit, static_argnames=("interpret",))
def kernel(x, W, b, interpret=False):
    Bsz = x.shape[0]
    d = np.arange(_TOTAL)
    sel_code = np.where((d[:, None] // _CODE_LENGTH) == np.arange(_NUM_TABLE)[None, :],
                        (2.0 ** (d % _CODE_LENGTH))[:, None], 0.0).astype(np.float32)
    sel_ext = (d[:, None] == (_CODE_LENGTH * np.arange(_NUM_TABLE))[None, :]
               ).astype(np.float32)
    grid = (Bsz // _BM,)
    code, score = pl.pallas_call(
        _fused_kernel,
        grid=grid,
        in_specs=[
            pl.BlockSpec((_BM, _HIDDEN), lambda i: (i, 0)),
            pl.BlockSpec((_HIDDEN, _TOTAL), lambda i: (0, 0)),
            pl.BlockSpec((1, _TOTAL), lambda i: (0, 0)),
            pl.BlockSpec((_TOTAL, _NUM_TABLE), lambda i: (0, 0)),
            pl.BlockSpec((_TOTAL, _NUM_TABLE), lambda i: (0, 0)),
        ],
        out_specs=[
            pl.BlockSpec((_BM, _NUM_TABLE), lambda i: (i, 0)),
            pl.BlockSpec((_BM, _NUM_TABLE), lambda i: (i, 0)),
        ],
        out_shape=[
            jax.ShapeDtypeStruct((Bsz, _NUM_TABLE), jnp.int32),
            jax.ShapeDtypeStruct((Bsz, _NUM_TABLE), jnp.float32),
        ],
        compiler_params=pltpu.CompilerParams(
            dimension_semantics=("parallel",)),
        interpret=interpret,
    )(x, W, b.reshape(1, _TOTAL), jnp.asarray(sel_code), jnp.asarray(sel_ext))
    return (code, score)


# speedup vs baseline: 1.3408x; 1.3408x over previous
import functools

import jax
import jax.numpy as jnp
import numpy as np
from jax.experimental import pallas as pl
from jax.experimental.pallas import tpu as pltpu

_NUM_TABLE = 32
_CODE_LENGTH = 16
_HIDDEN = 2048
_TOTAL = _NUM_TABLE * _CODE_LENGTH
_BM = 1024

_COLS = np.arange(_TOTAL)
_A, _REM = _COLS // 128, _COLS % 128
_Q, _T = _REM // 32, _REM % 32
_K = 4 * _A + _Q
_PERM = (_CODE_LENGTH * _T + _K).astype(np.int32)
_SELC = np.zeros((_TOTAL, _NUM_TABLE), np.float32)
_SELC[_COLS, _T] = 2.0 ** _K


def _fused_kernel(x_ref, w_ref, b_ref, selc_ref, code_ref, score_ref):
    z = jnp.dot(x_ref[...], w_ref[...], preferred_element_type=jnp.float32)
    z = z + b_ref[...]
    bits = (z > 0).astype(jnp.float32)
    codef = jax.lax.dot(bits, selc_ref[...],
                        preferred_element_type=jnp.float32)
    code_ref[...] = codef.astype(jnp.int32)
    za = jnp.abs(z)
    m = (za[:, 0:128] * za[:, 128:256]) * (za[:, 256:384] * za[:, 384:512])
    m = m * pltpu.roll(m, 96, 1)
    m = m * pltpu.roll(m, 64, 1)
    score_ref[...] = m[:, 0:_NUM_TABLE]


@functools.partial(jax.jit, static_argnames=("interpret",))
def kernel(x, W, b, interpret=False):
    Bsz = x.shape[0]
    perm = jnp.asarray(_PERM)
    W2 = jnp.take(W, perm, axis=1)
    b2 = jnp.take(b, perm).reshape(1, _TOTAL)
    grid = (Bsz // _BM,)
    code, score = pl.pallas_call(
        _fused_kernel,
        grid=grid,
        in_specs=[
            pl.BlockSpec((_BM, _HIDDEN), lambda i: (i, 0)),
            pl.BlockSpec((_HIDDEN, _TOTAL), lambda i: (0, 0)),
            pl.BlockSpec((1, _TOTAL), lambda i: (0, 0)),
            pl.BlockSpec((_TOTAL, _NUM_TABLE), lambda i: (0, 0)),
        ],
        out_specs=[
            pl.BlockSpec((_BM, _NUM_TABLE), lambda i: (i, 0)),
            pl.BlockSpec((_BM, _NUM_TABLE), lambda i: (i, 0)),
        ],
        out_shape=[
            jax.ShapeDtypeStruct((Bsz, _NUM_TABLE), jnp.int32),
            jax.ShapeDtypeStruct((Bsz, _NUM_TABLE), jnp.float32),
        ],
        compiler_params=pltpu.CompilerParams(
            dimension_semantics=("parallel",)),
        interpret=interpret,
    )(x, W2, b2, jnp.asarray(_SELC))
    return (code, score)
